# baseline (speedup 1.0000x reference)
import functools
import math

import jax
import jax.numpy as jnp
from jax import lax
from jax.experimental import pallas as pl
from jax.experimental.pallas import tpu as pltpu
from jax.experimental.pallas import tpu_sc as plsc

D_MODEL = 64
SCALE = math.sqrt(D_MODEL)
NUM_CORES = 2
NUM_SUBCORES = 16
NUM_WORKERS = NUM_CORES * NUM_SUBCORES
LANES = 16


@functools.partial(jax.jit, static_argnums=(2,))
def _gather_scale(x_flat, table, chunk):
    n = x_flat.shape[0]
    per_worker = n // NUM_WORKERS
    n_chunks = per_worker // chunk
    mesh = plsc.VectorSubcoreMesh(core_axis_name="c", subcore_axis_name="s")

    @functools.partial(
        pl.kernel,
        out_type=jax.ShapeDtypeStruct((n, D_MODEL), jnp.float32),
        mesh=mesh,
        scratch_types=[
            pltpu.VMEM((chunk,), jnp.int32),
            pltpu.VMEM((chunk, D_MODEL), jnp.float32),
            pltpu.SemaphoreType.DMA,
        ],
        compiler_params=pltpu.CompilerParams(use_tc_tiling_on_sc=False),
    )
    def k(x_hbm, table_hbm, out_hbm, idx_v, rows_v, sem):
        wid = lax.axis_index("s") * NUM_CORES + lax.axis_index("c")
        base = wid * per_worker

        def chunk_body(ci, carry):
            off = base + ci * chunk
            pltpu.sync_copy(x_hbm.at[pl.ds(off, chunk)], idx_v)
            pltpu.async_copy(table_hbm.at[idx_v], rows_v, sem).wait()

            def row_body(r, c2):
                for g in range(D_MODEL // LANES):
                    rows_v[r, pl.ds(g * LANES, LANES)] = (
                        rows_v[r, pl.ds(g * LANES, LANES)] * SCALE
                    )
                return c2

            lax.fori_loop(0, chunk, row_body, 0)
            pltpu.sync_copy(rows_v, out_hbm.at[pl.ds(off, chunk)])
            return carry

        lax.fori_loop(0, n_chunks, chunk_body, 0)

    return k(x_flat, table)


def kernel(x, table):
    b, s = x.shape
    x_flat = x.reshape(b * s).astype(jnp.int32)
    out = _gather_scale(x_flat, table, 512)
    return out.reshape(b, s, D_MODEL)


# speedup vs baseline: 1.1386x; 1.1386x over previous
import functools
import math

import jax
import jax.numpy as jnp
from jax import lax
from jax.experimental import pallas as pl
from jax.experimental.pallas import tpu as pltpu
from jax.experimental.pallas import tpu_sc as plsc

D_MODEL = 64
SCALE = math.sqrt(D_MODEL)
NUM_CORES = 2
NUM_SUBCORES = 16
NUM_WORKERS = NUM_CORES * NUM_SUBCORES
LANES = 16
NBUF = 2


@functools.partial(jax.jit, static_argnums=(2,))
def _gather_scale(x_flat, table, chunk):
    n = x_flat.shape[0]
    per_worker = n // NUM_WORKERS
    n_chunks = per_worker // chunk
    rounds = n_chunks // NBUF
    mesh = plsc.VectorSubcoreMesh(core_axis_name="c", subcore_axis_name="s")

    @functools.partial(
        pl.kernel,
        out_type=jax.ShapeDtypeStruct((n, D_MODEL), jnp.float32),
        mesh=mesh,
        scratch_types=[
            pltpu.VMEM((per_worker,), jnp.int32),
            [pltpu.VMEM((chunk, D_MODEL), jnp.float32) for _ in range(NBUF)],
            [pltpu.VMEM((chunk, D_MODEL), jnp.float32) for _ in range(NBUF)],
            [pltpu.SemaphoreType.DMA for _ in range(NBUF)],
            [pltpu.SemaphoreType.DMA for _ in range(NBUF)],
        ],
        compiler_params=pltpu.CompilerParams(use_tc_tiling_on_sc=False),
    )
    def k(x_hbm, table_hbm, out_hbm, idx_v, in_bufs, out_bufs, g_sems, s_sems):
        wid = lax.axis_index("s") * NUM_CORES + lax.axis_index("c")
        base = wid * per_worker

        pltpu.sync_copy(x_hbm.at[pl.ds(base, per_worker)], idx_v)

        def gather_start(ci, b):
            pltpu.async_copy(
                table_hbm.at[idx_v.at[pl.ds(ci * chunk, chunk)]],
                in_bufs[b],
                g_sems[b],
            )

        for b in range(NBUF):
            gather_start(b, b)

        def round_body(r, carry):
            for b in range(NBUF):
                ci = r * NBUF + b
                off = base + ci * chunk

                pltpu.make_async_copy(
                    table_hbm.at[idx_v.at[pl.ds(ci * chunk, chunk)]],
                    in_bufs[b],
                    g_sems[b],
                ).wait()

                @pl.when(r > 0)
                def _():
                    pltpu.make_async_copy(
                        out_bufs[b],
                        out_hbm.at[pl.ds(off, chunk)],
                        s_sems[b],
                    ).wait()

                @plsc.parallel_loop(0, chunk, unroll=8)
                def _(i):
                    for g in range(D_MODEL // LANES):
                        out_bufs[b][i, pl.ds(g * LANES, LANES)] = (
                            in_bufs[b][i, pl.ds(g * LANES, LANES)] * SCALE
                        )

                pltpu.async_copy(
                    out_bufs[b], out_hbm.at[pl.ds(off, chunk)], s_sems[b]
                )

                @pl.when(ci + NBUF < n_chunks)
                def _():
                    gather_start(ci + NBUF, b)

            return carry

        lax.fori_loop(0, rounds, round_body, 0)

        for b in range(NBUF):
            last_off = base + (rounds - 1) * NBUF * chunk + b * chunk
            pltpu.make_async_copy(
                out_bufs[b], out_hbm.at[pl.ds(last_off, chunk)], s_sems[b]
            ).wait()

    return k(x_flat, table)


def kernel(x, table):
    b, s = x.shape
    x_flat = x.reshape(b * s).astype(jnp.int32)
    out = _gather_scale(x_flat, table, 320)
    return out.reshape(b, s, D_MODEL)
